# baseline (speedup 1.0000x reference)
import jax
import jax.numpy as jnp
from jax.experimental import pallas as pl
from jax.experimental.pallas import tpu as pltpu

_BETA = 0.25
_N = 16384
_BLK = 1024
_L = 4
_K = 256
_D = 32

_F32 = jnp.float32
_BF16 = jnp.bfloat16

_ENC = [(768, 512), (512, 256), (256, 128), (128, 32)]
_DEC = [(32, 128), (128, 256), (256, 512), (512, 768)]


def _mm(a, b):
    return jnp.dot(a.astype(_BF16), b, preferred_element_type=_F32)


def _rqvae_block(x_ref,
                 ew0, eb0, ew1, eb1, ew2, eb2, ew3, eb3,
                 dw0, db0, dw1, db1, dw2, db2, dw3, db3,
                 cb_ref,
                 out_ref, idx_ref, loss_ref,
                 *scratch):
    i = pl.program_id(0)
    ws = scratch[:8]
    cbs = scratch[8]
    e2s = scratch[9]

    @pl.when(i == 0)
    def _prep():
        for w_ref, s_ref in zip((ew0, ew1, ew2, ew3, dw0, dw1, dw2, dw3), ws):
            s_ref[...] = w_ref[...].astype(_BF16)
        cb = cb_ref[...]
        hi = cb.astype(_BF16)
        t = cb - hi.astype(_F32)
        mid = t.astype(_BF16)
        lo = (t - mid.astype(_F32)).astype(_BF16)
        cbs[...] = jnp.concatenate([hi, mid, lo], axis=1)
        e2s[...] = jnp.sum(cb * cb, axis=2)

    h = x_ref[...]
    h = jnp.maximum(_mm(h, ws[0][...]) + eb0[...], 0.0)
    h = jnp.maximum(_mm(h, ws[1][...]) + eb1[...], 0.0)
    h = jnp.maximum(_mm(h, ws[2][...]) + eb2[...], 0.0)
    z = _mm(h, ws[3][...]) + eb3[...]

    residual = z
    x_q = jnp.zeros_like(z)
    loss_sum = jnp.float32(0.0)
    idx_cols = []
    lane_iota_f = jax.lax.broadcasted_iota(jnp.int32, (_BLK, _K), 1).astype(_F32)
    for l in range(_L):
        e_hi = cbs[l, :_K]
        e2 = e2s[l]
        r2 = jnp.sum(residual * residual, axis=1, keepdims=True)
        rd = jax.lax.dot_general(residual.astype(_BF16), e_hi,
                                 (((1,), (1,)), ((), ())),
                                 preferred_element_type=_F32)
        d = r2 + e2[None, :] - 2.0 * rd
        dmin = jnp.min(d, axis=1, keepdims=True)
        idx_f = jnp.min(jnp.where(d == dmin, lane_iota_f, float(_K)), axis=1)
        onehot = (lane_iota_f == idx_f[:, None]).astype(_BF16)
        q = jnp.dot(onehot, e_hi, preferred_element_type=_F32)
        q = q + jnp.dot(onehot, cbs[l, _K:2 * _K], preferred_element_type=_F32)
        q = q + jnp.dot(onehot, cbs[l, 2 * _K:], preferred_element_type=_F32)
        t = q - residual
        loss_sum = loss_sum + jnp.sum(t * t)
        q_st = residual + t
        x_q = x_q + q_st
        residual = residual - q_st
        idx_cols.append(idx_f.astype(jnp.int32))

    h = jnp.maximum(_mm(x_q, ws[4][...]) + db0[...], 0.0)
    h = jnp.maximum(_mm(h, ws[5][...]) + db1[...], 0.0)
    h = jnp.maximum(_mm(h, ws[6][...]) + db2[...], 0.0)
    out_ref[...] = _mm(h, ws[7][...]) + db3[...]

    idx_ref[...] = jnp.stack(idx_cols, axis=-1)

    nblk = pl.num_programs(0)
    prev = jnp.where(i == 0, 0.0, loss_ref[0, 0])
    acc = prev + loss_sum
    scale = (1.0 + _BETA) / (_L * _N * _D)
    loss_ref[0, 0] = jnp.where(i == nblk - 1, acc * scale, acc)


def kernel(x, ew0, eb0, ew1, eb1, ew2, eb2, ew3, eb3,
           dw0, db0, dw1, db1, dw2, db2, dw3, db3, cb):
    grid = (_N // _BLK,)

    def _row(i):
        return (i, 0)

    def _fixed(i):
        return (0, 0)

    def _fixed3(i):
        return (0, 0, 0)

    def _w(a):
        return pl.BlockSpec(a.shape, _fixed)

    biases = [b.reshape(1, -1) for b in (eb0, eb1, eb2, eb3, db0, db1, db2, db3)]
    eb0r, eb1r, eb2r, eb3r, db0r, db1r, db2r, db3r = biases

    in_specs = [
        pl.BlockSpec((_BLK, 768), _row),
        _w(ew0), _w(eb0r), _w(ew1), _w(eb1r), _w(ew2), _w(eb2r), _w(ew3), _w(eb3r),
        _w(dw0), _w(db0r), _w(dw1), _w(db1r), _w(dw2), _w(db2r), _w(dw3), _w(db3r),
        pl.BlockSpec(cb.shape, _fixed3),
    ]
    out_specs = (
        pl.BlockSpec((_BLK, 768), _row),
        pl.BlockSpec((_BLK, _L), _row),
        pl.BlockSpec((1, 1), _fixed, memory_space=pltpu.SMEM),
    )
    out_shapes = (
        jax.ShapeDtypeStruct((_N, 768), _F32),
        jax.ShapeDtypeStruct((_N, _L), jnp.int32),
        jax.ShapeDtypeStruct((1, 1), _F32),
    )
    scratch_shapes = (
        [pltpu.VMEM(s, _BF16) for s in _ENC]
        + [pltpu.VMEM(s, _BF16) for s in _DEC]
        + [pltpu.VMEM((_L, 3 * _K, _D), _BF16),
           pltpu.VMEM((_L, _K), _F32)]
    )

    out, idx, loss = pl.pallas_call(
        _rqvae_block,
        grid=grid,
        in_specs=in_specs,
        out_specs=out_specs,
        out_shape=out_shapes,
        scratch_shapes=scratch_shapes,
    )(x, ew0, eb0r, ew1, eb1r, ew2, eb2r, ew3, eb3r,
      dw0, db0r, dw1, db1r, dw2, db2r, dw3, db3r, cb)

    return out, loss[0, 0], idx


# speedup vs baseline: 1.0779x; 1.0779x over previous
import jax
import jax.numpy as jnp
from jax.experimental import pallas as pl
from jax.experimental.pallas import tpu as pltpu

_BETA = 0.25
_N = 16384
_BLK = 2048
_L = 4
_K = 256
_D = 32

_F32 = jnp.float32
_BF16 = jnp.bfloat16

_ENC = [(768, 512), (512, 256), (256, 128), (128, 32)]
_DEC = [(32, 128), (128, 256), (256, 512), (512, 768)]


def _mm(a, b):
    return jnp.dot(a.astype(_BF16), b, preferred_element_type=_F32)


def _rqvae_block(x_ref,
                 ew0, eb0, ew1, eb1, ew2, eb2, ew3, eb3,
                 dw0, db0, dw1, db1, dw2, db2, dw3, db3,
                 cb_ref,
                 out_ref, idx_ref, loss_ref,
                 *scratch):
    i = pl.program_id(0)
    ws = scratch[:8]
    cbs = scratch[8]
    e2s = scratch[9]

    @pl.when(i == 0)
    def _prep():
        for w_ref, s_ref in zip((ew0, ew1, ew2, ew3, dw0, dw1, dw2, dw3), ws):
            s_ref[...] = w_ref[...].astype(_BF16)
        cb = cb_ref[...]
        hi = cb.astype(_BF16)
        t = cb - hi.astype(_F32)
        mid = t.astype(_BF16)
        lo = (t - mid.astype(_F32)).astype(_BF16)
        cbs[...] = jnp.concatenate([hi, mid, lo], axis=1)
        e2s[...] = jnp.sum(cb * cb, axis=2)

    h = x_ref[...]
    h = jnp.maximum(_mm(h, ws[0][...]) + eb0[...], 0.0)
    h = jnp.maximum(_mm(h, ws[1][...]) + eb1[...], 0.0)
    h = jnp.maximum(_mm(h, ws[2][...]) + eb2[...], 0.0)
    z = _mm(h, ws[3][...]) + eb3[...]

    residual = z
    x_q = jnp.zeros_like(z)
    loss_sum = jnp.float32(0.0)
    idx_cols = []
    lane_iota_f = jax.lax.broadcasted_iota(jnp.int32, (_BLK, _K), 1).astype(_F32)
    for l in range(_L):
        e_hi = cbs[l, :_K]
        e2 = e2s[l]
        r2 = jnp.sum(residual * residual, axis=1, keepdims=True)
        rd = jax.lax.dot_general(residual.astype(_BF16), e_hi,
                                 (((1,), (1,)), ((), ())),
                                 preferred_element_type=_F32)
        d = r2 + e2[None, :] - 2.0 * rd
        dmin = jnp.min(d, axis=1, keepdims=True)
        idx_f = jnp.min(jnp.where(d == dmin, lane_iota_f, float(_K)), axis=1)
        onehot = (lane_iota_f == idx_f[:, None]).astype(_BF16)
        q = jnp.dot(onehot, e_hi, preferred_element_type=_F32)
        q = q + jnp.dot(onehot, cbs[l, _K:2 * _K], preferred_element_type=_F32)
        q = q + jnp.dot(onehot, cbs[l, 2 * _K:], preferred_element_type=_F32)
        t = q - residual
        loss_sum = loss_sum + jnp.sum(t * t)
        q_st = residual + t
        x_q = x_q + q_st
        residual = residual - q_st
        idx_cols.append(idx_f.astype(jnp.int32))

    h = jnp.maximum(_mm(x_q, ws[4][...]) + db0[...], 0.0)
    h = jnp.maximum(_mm(h, ws[5][...]) + db1[...], 0.0)
    h = jnp.maximum(_mm(h, ws[6][...]) + db2[...], 0.0)
    out_ref[...] = _mm(h, ws[7][...]) + db3[...]

    idx_ref[...] = jnp.stack(idx_cols, axis=-1)

    nblk = pl.num_programs(0)
    prev = jnp.where(i == 0, 0.0, loss_ref[0, 0])
    acc = prev + loss_sum
    scale = (1.0 + _BETA) / (_L * _N * _D)
    loss_ref[0, 0] = jnp.where(i == nblk - 1, acc * scale, acc)


def kernel(x, ew0, eb0, ew1, eb1, ew2, eb2, ew3, eb3,
           dw0, db0, dw1, db1, dw2, db2, dw3, db3, cb):
    grid = (_N // _BLK,)

    def _row(i):
        return (i, 0)

    def _fixed(i):
        return (0, 0)

    def _fixed3(i):
        return (0, 0, 0)

    def _w(a):
        return pl.BlockSpec(a.shape, _fixed)

    biases = [b.reshape(1, -1) for b in (eb0, eb1, eb2, eb3, db0, db1, db2, db3)]
    eb0r, eb1r, eb2r, eb3r, db0r, db1r, db2r, db3r = biases

    in_specs = [
        pl.BlockSpec((_BLK, 768), _row),
        _w(ew0), _w(eb0r), _w(ew1), _w(eb1r), _w(ew2), _w(eb2r), _w(ew3), _w(eb3r),
        _w(dw0), _w(db0r), _w(dw1), _w(db1r), _w(dw2), _w(db2r), _w(dw3), _w(db3r),
        pl.BlockSpec(cb.shape, _fixed3),
    ]
    out_specs = (
        pl.BlockSpec((_BLK, 768), _row),
        pl.BlockSpec((_BLK, _L), _row),
        pl.BlockSpec((1, 1), _fixed, memory_space=pltpu.SMEM),
    )
    out_shapes = (
        jax.ShapeDtypeStruct((_N, 768), _F32),
        jax.ShapeDtypeStruct((_N, _L), jnp.int32),
        jax.ShapeDtypeStruct((1, 1), _F32),
    )
    scratch_shapes = (
        [pltpu.VMEM(s, _BF16) for s in _ENC]
        + [pltpu.VMEM(s, _BF16) for s in _DEC]
        + [pltpu.VMEM((_L, 3 * _K, _D), _BF16),
           pltpu.VMEM((_L, _K), _F32)]
    )

    out, idx, loss = pl.pallas_call(
        _rqvae_block,
        grid=grid,
        in_specs=in_specs,
        out_specs=out_specs,
        out_shape=out_shapes,
        scratch_shapes=scratch_shapes,
    )(x, ew0, eb0r, ew1, eb1r, ew2, eb2r, ew3, eb3r,
      dw0, db0r, dw1, db1r, dw2, db2r, dw3, db3r, cb)

    return out, loss[0, 0], idx


# speedup vs baseline: 1.0869x; 1.0084x over previous
import jax
import jax.numpy as jnp
from jax.experimental import pallas as pl
from jax.experimental.pallas import tpu as pltpu

_BETA = 0.25
_N = 16384
_BLK = 2048
_L = 4
_K = 256
_D = 32

_F32 = jnp.float32
_BF16 = jnp.bfloat16

_ENC = [(768, 512), (512, 256), (256, 128), (128, 32)]
_DEC = [(32, 128), (128, 256), (256, 512), (512, 768)]


def _mm(a, b):
    return jnp.dot(a.astype(_BF16), b, preferred_element_type=_F32)


def _rqvae_block(x_ref,
                 ew0, eb0, ew1, eb1, ew2, eb2, ew3, eb3,
                 dw0, db0, dw1, db1, dw2, db2, dw3, db3,
                 cb_ref,
                 out_ref, idx_ref, loss_ref,
                 *scratch):
    i = pl.program_id(0)
    ws = scratch[:8]
    cbs = scratch[8]
    cbn2 = scratch[9]
    e2s = scratch[10]

    @pl.when(i == 0)
    def _prep():
        for w_ref, s_ref in zip((ew0, ew1, ew2, ew3, dw0, dw1, dw2, dw3), ws):
            s_ref[...] = w_ref[...].astype(_BF16)
        cb = cb_ref[...]
        hi = cb.astype(_BF16)
        t = cb - hi.astype(_F32)
        mid = t.astype(_BF16)
        lo = (t - mid.astype(_F32)).astype(_BF16)
        cbs[...] = jnp.concatenate([hi, mid, lo], axis=1)
        cbn2[...] = hi * jnp.bfloat16(-2.0)
        e2s[...] = jnp.sum(cb * cb, axis=2)

    h = x_ref[...]
    h = jnp.maximum(_mm(h, ws[0][...]) + eb0[...], 0.0)
    h = jnp.maximum(_mm(h, ws[1][...]) + eb1[...], 0.0)
    h = jnp.maximum(_mm(h, ws[2][...]) + eb2[...], 0.0)
    z = _mm(h, ws[3][...]) + eb3[...]

    residual = z
    x_q = jnp.zeros_like(z)
    loss_sum = jnp.float32(0.0)
    idx_cols = []
    lane_iota_f = jax.lax.broadcasted_iota(jnp.int32, (_BLK, _K), 1).astype(_F32)
    for l in range(_L):
        e_hi = cbs[l, :_K]
        e2 = e2s[l]
        r2 = jnp.sum(residual * residual, axis=1, keepdims=True)
        nrd2 = jax.lax.dot_general(residual.astype(_BF16), cbn2[l],
                                   (((1,), (1,)), ((), ())),
                                   preferred_element_type=_F32)
        d = (r2 + e2[None, :]) + nrd2
        dmin = jnp.min(d, axis=1, keepdims=True)
        idx_f = jnp.min(jnp.where(d == dmin, lane_iota_f, float(_K)), axis=1)
        onehot = (lane_iota_f == idx_f[:, None]).astype(_BF16)
        q = jnp.dot(onehot, e_hi, preferred_element_type=_F32)
        q = q + jnp.dot(onehot, cbs[l, _K:2 * _K], preferred_element_type=_F32)
        q = q + jnp.dot(onehot, cbs[l, 2 * _K:], preferred_element_type=_F32)
        t = q - residual
        loss_sum = loss_sum + jnp.sum(t * t)
        q_st = residual + t
        x_q = x_q + q_st
        residual = residual - q_st
        idx_cols.append(idx_f.astype(jnp.int32))

    h = jnp.maximum(_mm(x_q, ws[4][...]) + db0[...], 0.0)
    h = jnp.maximum(_mm(h, ws[5][...]) + db1[...], 0.0)
    h = jnp.maximum(_mm(h, ws[6][...]) + db2[...], 0.0)
    out_ref[...] = _mm(h, ws[7][...]) + db3[...]

    idx_ref[...] = jnp.stack(idx_cols, axis=-1)

    nblk = pl.num_programs(0)
    prev = jnp.where(i == 0, 0.0, loss_ref[0, 0])
    acc = prev + loss_sum
    scale = (1.0 + _BETA) / (_L * _N * _D)
    loss_ref[0, 0] = jnp.where(i == nblk - 1, acc * scale, acc)


def kernel(x, ew0, eb0, ew1, eb1, ew2, eb2, ew3, eb3,
           dw0, db0, dw1, db1, dw2, db2, dw3, db3, cb):
    grid = (_N // _BLK,)

    def _row(i):
        return (i, 0)

    def _fixed(i):
        return (0, 0)

    def _fixed3(i):
        return (0, 0, 0)

    def _w(a):
        return pl.BlockSpec(a.shape, _fixed)

    biases = [b.reshape(1, -1) for b in (eb0, eb1, eb2, eb3, db0, db1, db2, db3)]
    eb0r, eb1r, eb2r, eb3r, db0r, db1r, db2r, db3r = biases

    in_specs = [
        pl.BlockSpec((_BLK, 768), _row),
        _w(ew0), _w(eb0r), _w(ew1), _w(eb1r), _w(ew2), _w(eb2r), _w(ew3), _w(eb3r),
        _w(dw0), _w(db0r), _w(dw1), _w(db1r), _w(dw2), _w(db2r), _w(dw3), _w(db3r),
        pl.BlockSpec(cb.shape, _fixed3),
    ]
    out_specs = (
        pl.BlockSpec((_BLK, 768), _row),
        pl.BlockSpec((_BLK, _L), _row),
        pl.BlockSpec((1, 1), _fixed, memory_space=pltpu.SMEM),
    )
    out_shapes = (
        jax.ShapeDtypeStruct((_N, 768), _F32),
        jax.ShapeDtypeStruct((_N, _L), jnp.int32),
        jax.ShapeDtypeStruct((1, 1), _F32),
    )
    scratch_shapes = (
        [pltpu.VMEM(s, _BF16) for s in _ENC]
        + [pltpu.VMEM(s, _BF16) for s in _DEC]
        + [pltpu.VMEM((_L, 3 * _K, _D), _BF16),
           pltpu.VMEM((_L, _K, _D), _BF16),
           pltpu.VMEM((_L, _K), _F32)]
    )

    out, idx, loss = pl.pallas_call(
        _rqvae_block,
        grid=grid,
        in_specs=in_specs,
        out_specs=out_specs,
        out_shape=out_shapes,
        scratch_shapes=scratch_shapes,
    )(x, ew0, eb0r, ew1, eb1r, ew2, eb2r, ew3, eb3r,
      dw0, db0r, dw1, db1r, dw2, db2r, dw3, db3r, cb)

    return out, loss[0, 0], idx


# speedup vs baseline: 1.0876x; 1.0006x over previous
import jax
import jax.numpy as jnp
from jax.experimental import pallas as pl
from jax.experimental.pallas import tpu as pltpu

_BETA = 0.25
_N = 16384
_BLK = 2048
_L = 4
_K = 256
_D = 32

_F32 = jnp.float32
_BF16 = jnp.bfloat16

_ENC = [(768, 512), (512, 256), (256, 128), (128, 32)]
_DEC = [(32, 128), (128, 256), (256, 512), (512, 768)]


def _mm(a, b):
    return jnp.dot(a.astype(_BF16), b, preferred_element_type=_F32)


def _rqvae_block(x_ref,
                 ew0, eb0, ew1, eb1, ew2, eb2, ew3, eb3,
                 dw0, db0, dw1, db1, dw2, db2, dw3, db3,
                 cb_ref,
                 out_ref, idx_ref, loss_ref,
                 *scratch):
    i = pl.program_id(0)
    ws = scratch[:8]
    cbs = scratch[8]
    cbn2 = scratch[9]
    e2s = scratch[10]

    @pl.when(i == 0)
    def _prep():
        for w_ref, s_ref in zip((ew0, ew1, ew2, ew3, dw0, dw1, dw2, dw3), ws):
            s_ref[...] = w_ref[...].astype(_BF16)
        cb = cb_ref[...]
        hi = cb.astype(_BF16)
        t = cb - hi.astype(_F32)
        mid = t.astype(_BF16)
        lo = (t - mid.astype(_F32)).astype(_BF16)
        cbs[...] = jnp.concatenate([hi, mid, lo], axis=1)
        cbn2[...] = hi * jnp.bfloat16(-2.0)
        e2s[...] = jnp.sum(cb * cb, axis=2)

    h = x_ref[...]
    h = jnp.maximum(_mm(h, ws[0][...]) + eb0[...], 0.0)
    h = jnp.maximum(_mm(h, ws[1][...]) + eb1[...], 0.0)
    h = jnp.maximum(_mm(h, ws[2][...]) + eb2[...], 0.0)
    z = _mm(h, ws[3][...]) + eb3[...]

    residual = z
    x_q = jnp.zeros_like(z)
    loss_sum = jnp.float32(0.0)
    lane_iota_f = jax.lax.broadcasted_iota(jnp.int32, (_BLK, _K), 1).astype(_F32)
    for l in range(_L):
        e_hi = cbs[l, :_K]
        e2 = e2s[l]
        r2 = jnp.sum(residual * residual, axis=1, keepdims=True)
        nrd2 = jax.lax.dot_general(residual.astype(_BF16), cbn2[l],
                                   (((1,), (1,)), ((), ())),
                                   preferred_element_type=_F32)
        d = (r2 + e2[None, :]) + nrd2
        dmin = jnp.min(d, axis=1, keepdims=True)
        idx_f = jnp.min(jnp.where(d == dmin, lane_iota_f, float(_K)), axis=1)
        onehot = (lane_iota_f == idx_f[:, None]).astype(_BF16)
        q = jnp.dot(onehot, e_hi, preferred_element_type=_F32)
        q = q + jnp.dot(onehot, cbs[l, _K:2 * _K], preferred_element_type=_F32)
        q = q + jnp.dot(onehot, cbs[l, 2 * _K:], preferred_element_type=_F32)
        t = q - residual
        loss_sum = loss_sum + jnp.sum(t * t)
        q_st = residual + t
        x_q = x_q + q_st
        residual = residual - q_st
        idx_ref[:, l] = idx_f.astype(jnp.int32)

    h = jnp.maximum(_mm(x_q, ws[4][...]) + db0[...], 0.0)
    h = jnp.maximum(_mm(h, ws[5][...]) + db1[...], 0.0)
    h = jnp.maximum(_mm(h, ws[6][...]) + db2[...], 0.0)
    out_ref[...] = _mm(h, ws[7][...]) + db3[...]

    nblk = pl.num_programs(0)
    prev = jnp.where(i == 0, 0.0, loss_ref[0, 0])
    acc = prev + loss_sum
    scale = (1.0 + _BETA) / (_L * _N * _D)
    loss_ref[0, 0] = jnp.where(i == nblk - 1, acc * scale, acc)


def kernel(x, ew0, eb0, ew1, eb1, ew2, eb2, ew3, eb3,
           dw0, db0, dw1, db1, dw2, db2, dw3, db3, cb):
    grid = (_N // _BLK,)

    def _row(i):
        return (i, 0)

    def _fixed(i):
        return (0, 0)

    def _fixed3(i):
        return (0, 0, 0)

    def _w(a):
        return pl.BlockSpec(a.shape, _fixed)

    biases = [b.reshape(1, -1) for b in (eb0, eb1, eb2, eb3, db0, db1, db2, db3)]
    eb0r, eb1r, eb2r, eb3r, db0r, db1r, db2r, db3r = biases

    in_specs = [
        pl.BlockSpec((_BLK, 768), _row),
        _w(ew0), _w(eb0r), _w(ew1), _w(eb1r), _w(ew2), _w(eb2r), _w(ew3), _w(eb3r),
        _w(dw0), _w(db0r), _w(dw1), _w(db1r), _w(dw2), _w(db2r), _w(dw3), _w(db3r),
        pl.BlockSpec(cb.shape, _fixed3),
    ]
    out_specs = (
        pl.BlockSpec((_BLK, 768), _row),
        pl.BlockSpec((_BLK, _L), _row),
        pl.BlockSpec((1, 1), _fixed, memory_space=pltpu.SMEM),
    )
    out_shapes = (
        jax.ShapeDtypeStruct((_N, 768), _F32),
        jax.ShapeDtypeStruct((_N, _L), jnp.int32),
        jax.ShapeDtypeStruct((1, 1), _F32),
    )
    scratch_shapes = (
        [pltpu.VMEM(s, _BF16) for s in _ENC]
        + [pltpu.VMEM(s, _BF16) for s in _DEC]
        + [pltpu.VMEM((_L, 3 * _K, _D), _BF16),
           pltpu.VMEM((_L, _K, _D), _BF16),
           pltpu.VMEM((_L, _K), _F32)]
    )

    out, idx, loss = pl.pallas_call(
        _rqvae_block,
        grid=grid,
        in_specs=in_specs,
        out_specs=out_specs,
        out_shape=out_shapes,
        scratch_shapes=scratch_shapes,
    )(x, ew0, eb0r, ew1, eb1r, ew2, eb2r, ew3, eb3r,
      dw0, db0r, dw1, db1r, dw2, db2r, dw3, db3r, cb)

    return out, loss[0, 0], idx
